# Initial kernel scaffold; baseline (speedup 1.0000x reference)
import functools

import jax
import jax.numpy as jnp
from jax import lax
from jax.experimental import pallas as pl
from jax.experimental.pallas import tpu as pltpu
from jax.experimental.pallas import tpu_sc as plsc

T, C, H, DH = 2048, 768, 12, 64
D = 64
KVW = 2 * DH
NC, NS, L = 2, 16, 16
NW = NC * NS
TPW = T // NW

_mesh = plsc.VectorSubcoreMesh(
    core_axis_name="c", subcore_axis_name="s", num_cores=NC, num_subcores=NS)


def _mm_body(a_ref, b_ref, o_ref):
    o_ref[...] = lax.dot_general(
        a_ref[...], b_ref[...], (((1,), (1,)), ((), ())),
        preferred_element_type=jnp.float32)


def _matmul_bt(a, b, bm=512, bn=768):
    M, K = a.shape
    N = b.shape[0]
    return pl.pallas_call(
        _mm_body,
        grid=(M // bm, N // bn),
        in_specs=[pl.BlockSpec((bm, K), lambda i, j: (i, 0)),
                  pl.BlockSpec((bn, K), lambda i, j: (j, 0))],
        out_specs=pl.BlockSpec((bm, bn), lambda i, j: (i, j)),
        out_shape=jax.ShapeDtypeStruct((M, N), jnp.float32),
    )(a, b)


def _attn_body(q_hbm, kv_hbm, ng_hbm, out_hbm, qs, ns, kvg, sbuf, outs, sem):
    wid = lax.axis_index("s") * NC + lax.axis_index("c")
    t0 = wid * TPW
    iotas = [lax.broadcasted_iota(jnp.int32, (L,), 0) + c * L for c in range(4)]
    NEG = jnp.float32(-1e30)

    def head_body(h, carry):
        pltpu.sync_copy(q_hbm.at[pl.ds(t0, TPW), pl.ds(h * DH, DH)], qs)
        pltpu.sync_copy(ng_hbm.at[h, pl.ds(t0, TPW)], ns)

        def tok_body(ti, carry2):
            t = t0 + ti
            pltpu.async_copy(kv_hbm.at[h].at[ns.at[ti]], kvg, sem).wait()

            def dh_body(dh, acc):
                qv = jnp.full((L,), qs[ti, dh], jnp.float32)
                dhv = jnp.full((L,), dh, jnp.int32)
                return tuple(
                    acc[c] + qv * plsc.load_gather(kvg, [iotas[c], dhv])
                    for c in range(4))
            acc = lax.fori_loop(
                0, DH, dh_body,
                tuple(jnp.zeros((L,), jnp.float32) for _ in range(4)))

            masks = [ns[ti, pl.ds(c * L, L)] <= t for c in range(4)]
            mvecs = [jnp.where(masks[c], acc[c] * jnp.float32(0.125), NEG)
                     for c in range(4)]
            mx = jnp.max(jnp.maximum(jnp.maximum(mvecs[0], mvecs[1]),
                                     jnp.maximum(mvecs[2], mvecs[3])))
            mx = jnp.where(mx > jnp.float32(-5e29), mx, jnp.float32(0.0))
            evecs = [jnp.where(masks[c], jnp.exp(mvecs[c] - mx),
                               jnp.float32(0.0)) for c in range(4)]
            ssum = jnp.sum(evecs[0] + evecs[1] + evecs[2] + evecs[3])
            winv = jnp.float32(1.0) / jnp.maximum(ssum, jnp.float32(1e-9))
            for c in range(4):
                sbuf[pl.ds(c * L, L)] = evecs[c] * winv

            def j_body(j, oacc):
                wv = jnp.full((L,), sbuf[j], jnp.float32)
                return tuple(
                    oacc[c] + wv * kvg[j, pl.ds(DH + c * L, L)]
                    for c in range(4))
            oacc = lax.fori_loop(
                0, D, j_body,
                tuple(jnp.zeros((L,), jnp.float32) for _ in range(4)))
            for c in range(4):
                outs[ti, pl.ds(c * L, L)] = oacc[c]
            return carry2

        lax.fori_loop(0, TPW, tok_body, 0)
        pltpu.sync_copy(outs, out_hbm.at[pl.ds(t0, TPW), pl.ds(h * DH, DH)])
        return carry

    lax.fori_loop(0, H, head_body, 0)


_sc_attn = functools.partial(
    pl.kernel,
    out_type=jax.ShapeDtypeStruct((T, C), jnp.float32),
    mesh=_mesh,
    scratch_types=[
        pltpu.VMEM((TPW, DH), jnp.float32),
        pltpu.VMEM((TPW, D), jnp.int32),
        pltpu.VMEM((D, KVW), jnp.float32),
        pltpu.VMEM((D,), jnp.float32),
        pltpu.VMEM((TPW, DH), jnp.float32),
        pltpu.SemaphoreType.DMA,
    ],
)(_attn_body)


def kernel(x, neigh_idx, Wqkv, Wout):
    x2 = x[0]
    qkv = _matmul_bt(x2, Wqkv)
    kv = (qkv[:, C:]
          .reshape(T, 2, H, DH)
          .transpose(2, 0, 1, 3)
          .reshape(H, T, KVW))
    attn = _sc_attn(qkv, kv, neigh_idx.astype(jnp.int32))
    y = _matmul_bt(attn, Wout)
    return y[None]


# speedup vs baseline: 41.0305x; 41.0305x over previous
import functools

import jax
import jax.numpy as jnp
from jax import lax
from jax.experimental import pallas as pl
from jax.experimental.pallas import tpu as pltpu
from jax.experimental.pallas import tpu_sc as plsc

T, C, H, DH = 2048, 768, 12, 64
D = 64
KVW = 2 * DH
NC, NS, L = 2, 16, 16
NW = NC * NS
TPW = T // NW

_mesh = plsc.VectorSubcoreMesh(
    core_axis_name="c", subcore_axis_name="s", num_cores=NC, num_subcores=NS)


_DN = (((1,), (1,)), ((), ()))


def _qkv_body(x_ref, w_ref, q_ref, kv_ref):
    a = x_ref[...]
    for h in range(H):
        q_ref[h] = lax.dot_general(
            a, w_ref[pl.ds(h * DH, DH), :], _DN,
            preferred_element_type=jnp.float32)
        kv_ref[h, :, 0:DH] = lax.dot_general(
            a, w_ref[pl.ds(C + h * DH, DH), :], _DN,
            preferred_element_type=jnp.float32)
        kv_ref[h, :, DH:KVW] = lax.dot_general(
            a, w_ref[pl.ds(2 * C + h * DH, DH), :], _DN,
            preferred_element_type=jnp.float32)


def _qkv_proj(x, w, bm=256):
    return pl.pallas_call(
        _qkv_body,
        grid=(T // bm,),
        in_specs=[pl.BlockSpec((bm, C), lambda i: (i, 0)),
                  pl.BlockSpec((3 * C, C), lambda i: (0, 0))],
        out_specs=[pl.BlockSpec((H, bm, DH), lambda i: (0, i, 0)),
                   pl.BlockSpec((H, bm, KVW), lambda i: (0, i, 0))],
        out_shape=[jax.ShapeDtypeStruct((H, T, DH), jnp.float32),
                   jax.ShapeDtypeStruct((H, T, KVW), jnp.float32)],
    )(x, w)


def _oproj_body(a_ref, w_ref, o_ref):
    acc = jnp.zeros_like(o_ref)
    for h in range(H):
        acc = acc + lax.dot_general(
            a_ref[h], w_ref[:, pl.ds(h * DH, DH)], _DN,
            preferred_element_type=jnp.float32)
    o_ref[...] = acc


def _out_proj(attn, w, bm=256):
    return pl.pallas_call(
        _oproj_body,
        grid=(T // bm,),
        in_specs=[pl.BlockSpec((H, bm, DH), lambda i: (0, i, 0)),
                  pl.BlockSpec((C, C), lambda i: (0, 0))],
        out_specs=pl.BlockSpec((bm, C), lambda i: (i, 0)),
        out_shape=jax.ShapeDtypeStruct((T, C), jnp.float32),
    )(attn, w)


def _attn_body(q_hbm, kv_hbm, ng_hbm, out_hbm, qs, ns, kvg, outs, sem):
    wid = lax.axis_index("s") * NC + lax.axis_index("c")
    t0 = wid * TPW
    iota = lax.broadcasted_iota(jnp.int32, (L,), 0)
    iotas = [iota + c * L for c in range(4)]
    NEG = jnp.float32(-1e30)

    def head_body(h, carry):
        pltpu.sync_copy(q_hbm.at[h, pl.ds(t0, TPW)], qs)
        pltpu.sync_copy(ng_hbm.at[h, pl.ds(t0, TPW)], ns)

        def tok_body(ti, carry2):
            t = t0 + ti
            pltpu.async_copy(kv_hbm.at[h].at[ns.at[ti]], kvg, sem).wait()

            qv = [qs[ti, pl.ds(c * L, L)] for c in range(4)]
            acc = []
            for wc in range(4):
                sv = jnp.zeros((L,), jnp.float32)
                for lane in range(L):
                    j = wc * L + lane
                    d0 = kvg[j, pl.ds(0, L)] * qv[0]
                    for c in range(1, 4):
                        d0 = d0 + kvg[j, pl.ds(c * L, L)] * qv[c]
                    s = jnp.sum(d0)
                    sv = jnp.where(iota == jnp.int32(lane),
                                   jnp.full((L,), s, jnp.float32), sv)
                acc.append(sv)

            masks = [ns[ti, pl.ds(c * L, L)] <= t for c in range(4)]
            mvecs = [jnp.where(masks[c], acc[c] * jnp.float32(0.125), NEG)
                     for c in range(4)]
            mx = jnp.max(jnp.maximum(jnp.maximum(mvecs[0], mvecs[1]),
                                     jnp.maximum(mvecs[2], mvecs[3])))
            mx = jnp.where(mx > jnp.float32(-5e29), mx, jnp.float32(0.0))
            evecs = [jnp.where(masks[c], jnp.exp(mvecs[c] - mx),
                               jnp.float32(0.0)) for c in range(4)]
            ssum = jnp.sum(evecs[0] + evecs[1] + evecs[2] + evecs[3])
            winv = jnp.ones((L,), jnp.float32) / jnp.maximum(
                jnp.full((L,), ssum, jnp.float32), jnp.float32(1e-9))
            wvecs = [evecs[c] * winv for c in range(4)]

            oacc = [jnp.zeros((L,), jnp.float32) for _ in range(4)]
            for wc in range(4):
                for lane in range(L):
                    j = wc * L + lane
                    wb = jnp.full((L,), wvecs[wc][lane], jnp.float32)
                    for c in range(4):
                        oacc[c] = oacc[c] + wb * kvg[j, pl.ds(DH + c * L, L)]
            for c in range(4):
                outs[ti, pl.ds(c * L, L)] = oacc[c]
            return carry2

        lax.fori_loop(0, TPW, tok_body, 0)
        pltpu.sync_copy(outs, out_hbm.at[h, pl.ds(t0, TPW)])
        return carry

    lax.fori_loop(0, H, head_body, 0)


_sc_attn = functools.partial(
    pl.kernel,
    out_type=jax.ShapeDtypeStruct((H, T, DH), jnp.float32),
    mesh=_mesh,
    compiler_params=pltpu.CompilerParams(
        needs_layout_passes=False, use_tc_tiling_on_sc=False),
    scratch_types=[
        pltpu.VMEM((TPW, DH), jnp.float32),
        pltpu.VMEM((TPW, D), jnp.int32),
        pltpu.VMEM((D, KVW), jnp.float32),
        pltpu.VMEM((TPW, DH), jnp.float32),
        pltpu.SemaphoreType.DMA,
    ],
)(_attn_body)


def kernel(x, neigh_idx, Wqkv, Wout):
    x2 = x[0]
    q, kv = _qkv_proj(x2, Wqkv)
    attn = _sc_attn(q, kv, neigh_idx.astype(jnp.int32))
    y = _out_proj(attn, Wout)
    return y[None]


# speedup vs baseline: 67.1543x; 1.6367x over previous
import functools

import jax
import jax.numpy as jnp
from jax import lax
from jax.experimental import pallas as pl
from jax.experimental.pallas import tpu as pltpu
from jax.experimental.pallas import tpu_sc as plsc

T, C, H, DH = 2048, 768, 12, 64
D = 64
KVW = 2 * DH
NC, NS, L = 2, 16, 16
NW = NC * NS
TPW = T // NW

_mesh = plsc.VectorSubcoreMesh(
    core_axis_name="c", subcore_axis_name="s", num_cores=NC, num_subcores=NS)


_DN = (((1,), (1,)), ((), ()))


def _qkv_body(x_ref, w_ref, q_ref, kv_ref):
    a = x_ref[...]
    for h in range(H):
        q_ref[h] = lax.dot_general(
            a, w_ref[pl.ds(h * DH, DH), :], _DN,
            preferred_element_type=jnp.float32)
        kv_ref[h, :, 0:DH] = lax.dot_general(
            a, w_ref[pl.ds(C + h * DH, DH), :], _DN,
            preferred_element_type=jnp.float32)
        kv_ref[h, :, DH:KVW] = lax.dot_general(
            a, w_ref[pl.ds(2 * C + h * DH, DH), :], _DN,
            preferred_element_type=jnp.float32)


def _qkv_proj(x, w, bm=256):
    return pl.pallas_call(
        _qkv_body,
        grid=(T // bm,),
        in_specs=[pl.BlockSpec((bm, C), lambda i: (i, 0)),
                  pl.BlockSpec((3 * C, C), lambda i: (0, 0))],
        out_specs=[pl.BlockSpec((H, bm, DH), lambda i: (0, i, 0)),
                   pl.BlockSpec((H, bm, KVW), lambda i: (0, i, 0))],
        out_shape=[jax.ShapeDtypeStruct((H, T, DH), jnp.float32),
                   jax.ShapeDtypeStruct((H, T, KVW), jnp.float32)],
    )(x, w)


def _oproj_body(a_ref, w_ref, o_ref):
    acc = jnp.zeros_like(o_ref)
    for h in range(H):
        acc = acc + lax.dot_general(
            a_ref[h], w_ref[:, pl.ds(h * DH, DH)], _DN,
            preferred_element_type=jnp.float32)
    o_ref[...] = acc


def _out_proj(attn, w, bm=256):
    return pl.pallas_call(
        _oproj_body,
        grid=(T // bm,),
        in_specs=[pl.BlockSpec((H, bm, DH), lambda i: (0, i, 0)),
                  pl.BlockSpec((C, C), lambda i: (0, 0))],
        out_specs=pl.BlockSpec((bm, C), lambda i: (i, 0)),
        out_shape=jax.ShapeDtypeStruct((T, C), jnp.float32),
    )(attn, w)


def _attn_body(q_hbm, kv_hbm, ng_hbm, out_hbm, qs, ns, kvga, kvgb, outs,
               sema, semb):
    wid = lax.axis_index("s") * NC + lax.axis_index("c")
    t0 = wid * TPW
    iota = lax.broadcasted_iota(jnp.int32, (L,), 0)
    NEG = jnp.float32(-1e30)

    def compute_token(ti, kvg):
        t = t0 + ti
        qv = [qs[ti, pl.ds(c * L, L)] for c in range(4)]
        acc = []
        for wc in range(4):
            sv = jnp.zeros((L,), jnp.float32)
            for lane in range(L):
                j = wc * L + lane
                d0 = kvg[j, pl.ds(0, L)] * qv[0]
                for c in range(1, 4):
                    d0 = d0 + kvg[j, pl.ds(c * L, L)] * qv[c]
                s = jnp.sum(d0)
                sv = jnp.where(iota == jnp.int32(lane),
                               jnp.full((L,), s, jnp.float32), sv)
            acc.append(sv)

        masks = [ns[ti, pl.ds(c * L, L)] <= t for c in range(4)]
        mvecs = [jnp.where(masks[c], acc[c] * jnp.float32(0.125), NEG)
                 for c in range(4)]
        mx = jnp.max(jnp.maximum(jnp.maximum(mvecs[0], mvecs[1]),
                                 jnp.maximum(mvecs[2], mvecs[3])))
        mx = jnp.where(mx > jnp.float32(-5e29), mx, jnp.float32(0.0))
        evecs = [jnp.where(masks[c], jnp.exp(mvecs[c] - mx),
                           jnp.float32(0.0)) for c in range(4)]
        ssum = jnp.sum(evecs[0] + evecs[1] + evecs[2] + evecs[3])
        winv = jnp.ones((L,), jnp.float32) / jnp.maximum(
            jnp.full((L,), ssum, jnp.float32), jnp.float32(1e-9))
        wvecs = [evecs[c] * winv for c in range(4)]

        oacc = [jnp.zeros((L,), jnp.float32) for _ in range(4)]
        for wc in range(4):
            for lane in range(L):
                j = wc * L + lane
                wb = jnp.full((L,), wvecs[wc][lane], jnp.float32)
                for c in range(4):
                    oacc[c] = oacc[c] + wb * kvg[j, pl.ds(DH + c * L, L)]
        for c in range(4):
            outs[ti, pl.ds(c * L, L)] = oacc[c]

    def head_body(h, carry):
        pltpu.sync_copy(q_hbm.at[h, pl.ds(t0, TPW)], qs)
        pltpu.sync_copy(ng_hbm.at[h, pl.ds(t0, TPW)], ns)

        def gather(ti, buf, sem):
            pltpu.make_async_copy(kv_hbm.at[h].at[ns.at[ti]], buf,
                                  sem).start()

        gather(0, kvga, sema)

        def tok_body(ti2, carry2):
            ta = 2 * ti2
            gather(ta + 1, kvgb, semb)
            pltpu.make_async_copy(kv_hbm.at[h].at[ns.at[ta]], kvga,
                                  sema).wait()
            compute_token(ta, kvga)
            gather(jnp.minimum(ta + 2, TPW - 1), kvga, sema)
            pltpu.make_async_copy(kv_hbm.at[h].at[ns.at[ta + 1]], kvgb,
                                  semb).wait()
            compute_token(ta + 1, kvgb)
            return carry2

        lax.fori_loop(0, TPW // 2, tok_body, 0)
        pltpu.make_async_copy(kv_hbm.at[h].at[ns.at[TPW - 1]], kvga,
                              sema).wait()
        pltpu.sync_copy(outs, out_hbm.at[h, pl.ds(t0, TPW)])
        return carry

    lax.fori_loop(0, H, head_body, 0)


_sc_attn = functools.partial(
    pl.kernel,
    out_type=jax.ShapeDtypeStruct((H, T, DH), jnp.float32),
    mesh=_mesh,
    compiler_params=pltpu.CompilerParams(
        needs_layout_passes=False, use_tc_tiling_on_sc=False),
    scratch_types=[
        pltpu.VMEM((TPW, DH), jnp.float32),
        pltpu.VMEM((TPW, D), jnp.int32),
        pltpu.VMEM((D, KVW), jnp.float32),
        pltpu.VMEM((D, KVW), jnp.float32),
        pltpu.VMEM((TPW, DH), jnp.float32),
        pltpu.SemaphoreType.DMA,
        pltpu.SemaphoreType.DMA,
    ],
)(_attn_body)


def kernel(x, neigh_idx, Wqkv, Wout):
    x2 = x[0]
    q, kv = _qkv_proj(x2, Wqkv)
    attn = _sc_attn(q, kv, neigh_idx.astype(jnp.int32))
    y = _out_proj(attn, Wout)
    return y[None]
